# Initial kernel scaffold; baseline (speedup 1.0000x reference)
import jax
import jax.numpy as jnp
from jax.experimental import pallas as pl


def kernel(x, edge_index, gambling_scores, W1, as1, ad1, W2, as2, ad2, W3, as3, ad3, p1w, p1b, p2w, p2b, p3w, p3b, n1g, n1b, n2g, n2b, n3g, n3b, gew, geb, c1w, c1b, c2w, c2b):
    raise NotImplementedError("write your pallas kernel here")


# speedup vs baseline: 1.0037x; 1.0037x over previous
import functools

import jax
import jax.numpy as jnp
from jax.experimental import pallas as pl
from jax.experimental.pallas import tpu as pltpu

N = 10000
E = 320000
HID = 64
HEADS = 8
NEG_SLOPE = 0.2
GW = 1.0


def _mm_kernel(x_ref, w_ref, o_ref):
    o_ref[...] = jax.lax.dot_general(
        x_ref[...], w_ref[...], (((1,), (1,)), ((), ())),
        preferred_element_type=jnp.float32)


def _matmul_wt(x, w):
    n, d = x.shape
    k = w.shape[0]
    bn = 1000
    return pl.pallas_call(
        _mm_kernel,
        grid=(n // bn,),
        in_specs=[
            pl.BlockSpec((bn, d), lambda i: (i, 0)),
            pl.BlockSpec((k, d), lambda i: (0, 0)),
        ],
        out_specs=pl.BlockSpec((bn, k), lambda i: (i, 0)),
        out_shape=jax.ShapeDtypeStruct((n, k), jnp.float32),
    )(x, w)


def _segment_softmax(alpha, seg, num_segments):
    amax = jax.ops.segment_max(alpha, seg, num_segments=num_segments)
    amax = jnp.where(jnp.isfinite(amax), amax, 0.0)
    ex = jnp.exp(alpha - amax[seg])
    denom = jax.ops.segment_sum(ex, seg, num_segments=num_segments)
    return ex / (denom[seg] + 1e-16)


def _gat_conv(x, edge_index, gs, W, att_src, att_dst):
    n = x.shape[0]
    h = _matmul_wt(x, W).reshape(n, HEADS, -1)
    src = edge_index[0]
    dst = edge_index[1]
    x_j = h[src]
    x_i = h[dst]
    alpha_src = (x_i * att_src).sum(axis=-1)
    alpha_dst = (x_j * att_dst).sum(axis=-1)
    alpha = jax.nn.leaky_relu(alpha_src + alpha_dst, NEG_SLOPE)
    alpha = _segment_softmax(alpha, dst, n)
    weighted = 1.0 + GW * gs[src][:, None]
    alpha = alpha * weighted
    msg = x_j * alpha[..., None]
    out = jax.ops.segment_sum(msg, dst, num_segments=n)
    return out.mean(axis=1)


def _layernorm(x, g, b, eps=1e-5):
    mu = x.mean(axis=-1, keepdims=True)
    var = ((x - mu) ** 2).mean(axis=-1, keepdims=True)
    return (x - mu) / jnp.sqrt(var + eps) * g + b


def kernel(x, edge_index, gambling_scores, W1, as1, ad1, W2, as2, ad2, W3, as3, ad3, p1w, p1b, p2w, p2b, p3w, p3b, n1g, n1b, n2g, n2b, n3g, n3b, gew, geb, c1w, c1b, c2w, c2b):
    gemb = gambling_scores[:, None] @ gew.T + geb
    identity = x
    h = _gat_conv(x, edge_index, gambling_scores, W1, as1, ad1)
    h = _layernorm(h + gemb, n1g, n1b)
    identity = _matmul_wt(identity, p1w) + p1b
    h = jax.nn.relu(h + identity)
    identity = h
    h2 = _gat_conv(h, edge_index, gambling_scores, W2, as2, ad2)
    h2 = _layernorm(h2 + gemb, n2g, n2b)
    identity = _matmul_wt(identity, p2w) + p2b
    h2 = jax.nn.relu(h2 + identity)
    identity = h2
    h3 = _gat_conv(h2, edge_index, gambling_scores, W3, as3, ad3)
    h3 = _layernorm(h3 + gemb, n3g, n3b)
    identity = _matmul_wt(identity, p3w) + p3b
    h3 = jax.nn.relu(h3 + identity)
    pooled = h3.mean(axis=0, keepdims=True)
    z = jax.nn.relu(pooled @ c1w.T + c1b)
    z = z @ c2w.T + c2b
    return jax.nn.sigmoid(z)


# speedup vs baseline: 30.1005x; 29.9895x over previous
import functools

import jax
import jax.numpy as jnp
from jax import lax
from jax.experimental import pallas as pl
from jax.experimental.pallas import tpu as pltpu
from jax.experimental.pallas import tpu_sc as plsc

N = 10000
E = 320000
HID = 64
HEADS = 8
NH = HEADS * HID
NEG_SLOPE = 0.2

NC = 2
NS = 16
NW = NC * NS
EW = E // NW
C1 = 80
C2 = 80
NCH1 = EW // C1
NCH2 = EW // C2
BN = 1000

_MESH = plsc.VectorSubcoreMesh(
    core_axis_name="c", subcore_axis_name="s", num_cores=NC, num_subcores=NS)


def _sc_pass1(ab_hbm, ba_hbm, gs_hbm, src_hbm, dst_hbm,
              denom_out, v_out,
              gs_v, src_v, dst_v, r1, r2, vbuf, denom_t, sem):
    c_idx = lax.axis_index("c")
    s_idx = lax.axis_index("s")
    wid = s_idx * NC + c_idx
    iota = lax.iota(jnp.int32, 16)
    headmask = iota < 8

    pltpu.sync_copy(gs_hbm, gs_v)

    def zero(i, _):
        denom_t[pl.ds(i * 16, 16)] = jnp.zeros((16,), jnp.float32)
        return _

    lax.fori_loop(0, N * HEADS // 16, zero, None)

    def chunk(g, _):
        e0 = wid * EW + g * C1
        pltpu.sync_copy(src_hbm.at[pl.ds(e0, C1)], src_v)
        pltpu.sync_copy(dst_hbm.at[pl.ds(e0, C1)], dst_v)
        ca = pltpu.async_copy(ab_hbm.at[dst_v], r1, sem)
        pltpu.sync_copy(ba_hbm.at[src_v], r2)
        ca.wait()

        def edge(e, _):
            s = r1[e, :] + r2[e, :]
            u = jnp.exp(jnp.maximum(s, NEG_SLOPE * s))
            esplat = jnp.full((16,), e, jnp.int32)
            srcsplat = plsc.load_gather(src_v, [esplat])
            gsv = plsc.load_gather(gs_v, [srcsplat])
            vbuf[e, :] = u * (1.0 + gsv)
            dstsplat = plsc.load_gather(dst_v, [esplat])
            plsc.addupdate_scatter(denom_t, [dstsplat * 8 + iota], u,
                                   mask=headmask)
            return _

        lax.fori_loop(0, C1, edge, None)
        pltpu.sync_copy(vbuf, v_out.at[pl.ds(e0, C1)])
        return _

    lax.fori_loop(0, NCH1, chunk, None)
    pltpu.sync_copy(denom_t, denom_out.at[wid])


def _sc_pass2(hf_hbm, v_hbm, d16_hbm, src_hbm, dst_hbm, z64_hbm,
              acc_out,
              src_v, dst_v, vbuf, dr, cbuf, hfb, msg, acc_sh, sem):
    c_idx = lax.axis_index("c")
    s_idx = lax.axis_index("s")
    wid = s_idx * NC + c_idx
    iota = lax.iota(jnp.int32, 16)

    pltpu.sync_copy(z64_hbm.at[pl.ds(s_idx * 625, 625)],
                    acc_sh.at[pl.ds(s_idx * 625, 625)])
    plsc.subcore_barrier()

    def chunk(g, _):
        e0 = wid * EW + g * C2
        pltpu.sync_copy(src_hbm.at[pl.ds(e0, C2)], src_v)
        pltpu.sync_copy(dst_hbm.at[pl.ds(e0, C2)], dst_v)
        cg = pltpu.async_copy(hf_hbm.at[src_v], hfb, sem)
        pltpu.sync_copy(v_hbm.at[pl.ds(e0, C2)], vbuf)
        pltpu.sync_copy(d16_hbm.at[dst_v], dr)

        def coeff(e, _):
            vv = vbuf[e, :]
            dv = dr[e, :]
            c = vv / ((dv + 1e-16) * float(HEADS))
            cbuf[pl.ds(e * 16, 16)] = c
            return _

        lax.fori_loop(0, C2, coeff, None)
        cg.wait()

        def edge(e, _):
            m = [jnp.zeros((16,), jnp.float32) for _ in range(4)]
            for h in range(HEADS):
                cb = plsc.load_gather(
                    cbuf, [jnp.full((16,), e * 16 + h, jnp.int32)])
                for j in range(4):
                    hv = hfb[e, pl.ds(h * HID + j * 16, 16)]
                    m[j] = m[j] + cb * hv
            for j in range(4):
                msg[e, pl.ds(j * 16, 16)] = m[j]
            return _

        lax.fori_loop(0, C2, edge, None)
        pltpu.sync_copy(msg, acc_sh.at[dst_v], add=True)
        return _

    lax.fori_loop(0, NCH2, chunk, None)
    plsc.subcore_barrier()
    pltpu.sync_copy(acc_sh.at[pl.ds(s_idx * 625, 625)],
                    acc_out.at[c_idx, pl.ds(s_idx * 625, 625)])


def _make_pass1():
    return functools.partial(
        pl.kernel,
        out_type=[jax.ShapeDtypeStruct((NW, N * HEADS), jnp.float32),
                  jax.ShapeDtypeStruct((E, 16), jnp.float32)],
        mesh=_MESH,
        scratch_types=[
            pltpu.VMEM((N,), jnp.float32),
            pltpu.VMEM((C1,), jnp.int32),
            pltpu.VMEM((C1,), jnp.int32),
            pltpu.VMEM((C1, 16), jnp.float32),
            pltpu.VMEM((C1, 16), jnp.float32),
            pltpu.VMEM((C1, 16), jnp.float32),
            pltpu.VMEM((N * HEADS,), jnp.float32),
            pltpu.SemaphoreType.DMA,
        ],
        compiler_params=pltpu.CompilerParams(needs_layout_passes=False, use_tc_tiling_on_sc=False),
    )(_sc_pass1)


def _make_pass2():
    return functools.partial(
        pl.kernel,
        out_type=[jax.ShapeDtypeStruct((NC, N, HID), jnp.float32)],
        mesh=_MESH,
        scratch_types=[
            pltpu.VMEM((C2,), jnp.int32),
            pltpu.VMEM((C2,), jnp.int32),
            pltpu.VMEM((C2, 16), jnp.float32),
            pltpu.VMEM((C2, 16), jnp.float32),
            pltpu.VMEM((C2 * 16,), jnp.float32),
            pltpu.VMEM((C2, NH), jnp.float32),
            pltpu.VMEM((C2, HID), jnp.float32),
            pltpu.VMEM_SHARED((N, HID), jnp.float32),
            pltpu.SemaphoreType.DMA,
        ],
        compiler_params=pltpu.CompilerParams(needs_layout_passes=False, use_tc_tiling_on_sc=False),
    )(_sc_pass2)


def _tc_pre_kernel(x_ref, w1_ref, bs_ref, bd_ref, gs_ref, gewt_ref, geb_ref,
                   h_ref, ab_ref, ba_ref, gemb_ref):
    h = lax.dot_general(x_ref[...], w1_ref[...], (((1,), (1,)), ((), ())),
                        preferred_element_type=jnp.float32)
    h_ref[...] = h
    a1 = jnp.dot(h, bs_ref[...], preferred_element_type=jnp.float32)
    a2 = jnp.dot(h, bd_ref[...], preferred_element_type=jnp.float32)
    ab_ref[...] = jnp.concatenate([a1, a2], axis=-1)
    ba_ref[...] = jnp.concatenate([a2, a1], axis=-1)
    gemb_ref[...] = jnp.dot(gs_ref[...], gewt_ref[...],
                            preferred_element_type=jnp.float32) + geb_ref[...]


def _tc_dcomb_kernel(dp_ref, d16_ref):
    d = jnp.sum(dp_ref[...], axis=0)
    d16_ref[...] = jnp.concatenate([d, jnp.ones_like(d)], axis=-1)


def _tc_dcomb(denom_p):
    return pl.pallas_call(
        _tc_dcomb_kernel,
        grid=(N // BN,),
        in_specs=[pl.BlockSpec((NW, BN, HEADS), lambda i: (0, i, 0))],
        out_specs=pl.BlockSpec((BN, 16), lambda i: (i, 0)),
        out_shape=jax.ShapeDtypeStruct((N, 16), jnp.float32),
    )(denom_p)


def _tc_mid_kernel(acc_ref, gemb_ref, xp_ref, pw_ref, pb_ref, ng_ref, nb_ref,
                   wn_ref, bs_ref, bd_ref,
                   ho_ref, hn_ref, ab_ref, ba_ref):
    s = acc_ref[0] + acc_ref[1] + gemb_ref[...]
    mu = jnp.mean(s, axis=-1, keepdims=True)
    var = jnp.mean((s - mu) ** 2, axis=-1, keepdims=True)
    ln = (s - mu) * lax.rsqrt(var + 1e-5) * ng_ref[...] + nb_ref[...]
    ident = lax.dot_general(xp_ref[...], pw_ref[...], (((1,), (1,)), ((), ())),
                            preferred_element_type=jnp.float32) + pb_ref[...]
    ho = jnp.maximum(ln + ident, 0.0)
    ho_ref[...] = ho
    hn = lax.dot_general(ho, wn_ref[...], (((1,), (1,)), ((), ())),
                         preferred_element_type=jnp.float32)
    hn_ref[...] = hn
    a1 = jnp.dot(hn, bs_ref[...], preferred_element_type=jnp.float32)
    a2 = jnp.dot(hn, bd_ref[...], preferred_element_type=jnp.float32)
    ab_ref[...] = jnp.concatenate([a1, a2], axis=-1)
    ba_ref[...] = jnp.concatenate([a2, a1], axis=-1)


def _tc_last_kernel(acc_ref, gemb_ref, xp_ref, pw_ref, pb_ref, ng_ref, nb_ref,
                    ho_ref):
    s = acc_ref[0] + acc_ref[1] + gemb_ref[...]
    mu = jnp.mean(s, axis=-1, keepdims=True)
    var = jnp.mean((s - mu) ** 2, axis=-1, keepdims=True)
    ln = (s - mu) * lax.rsqrt(var + 1e-5) * ng_ref[...] + nb_ref[...]
    ident = lax.dot_general(xp_ref[...], pw_ref[...], (((1,), (1,)), ((), ())),
                            preferred_element_type=jnp.float32) + pb_ref[...]
    ho_ref[...] = jnp.maximum(ln + ident, 0.0)


def _tc_final_kernel(h_ref, c1w_ref, c1b_ref, c2w_ref, c2b_ref, o_ref):
    pooled = jnp.mean(h_ref[...], axis=0, keepdims=True)
    z = lax.dot_general(pooled, c1w_ref[...], (((1,), (1,)), ((), ())),
                        preferred_element_type=jnp.float32) + c1b_ref[...]
    z = jnp.maximum(z, 0.0)
    z = jnp.sum(z * c2w_ref[...], axis=-1, keepdims=True) + c2b_ref[...]
    o_ref[...] = 1.0 / (1.0 + jnp.exp(-z))


def _full(shape):
    return pl.BlockSpec(shape, lambda *_: tuple(0 for _ in shape))


def _tc_pre(x, w1, bs, bd, gs2d, gewt, geb2d):
    din = x.shape[1]
    return pl.pallas_call(
        _tc_pre_kernel,
        grid=(N // BN,),
        in_specs=[
            pl.BlockSpec((BN, din), lambda i: (i, 0)),
            _full((NH, din)), _full((NH, HEADS)), _full((NH, HEADS)),
            pl.BlockSpec((BN, 1), lambda i: (i, 0)),
            _full((1, HID)), _full((1, HID)),
        ],
        out_specs=[
            pl.BlockSpec((BN, NH), lambda i: (i, 0)),
            pl.BlockSpec((BN, 16), lambda i: (i, 0)),
            pl.BlockSpec((BN, 16), lambda i: (i, 0)),
            pl.BlockSpec((BN, HID), lambda i: (i, 0)),
        ],
        out_shape=[
            jax.ShapeDtypeStruct((N, NH), jnp.float32),
            jax.ShapeDtypeStruct((N, 16), jnp.float32),
            jax.ShapeDtypeStruct((N, 16), jnp.float32),
            jax.ShapeDtypeStruct((N, HID), jnp.float32),
        ],
    )(x, w1, bs, bd, gs2d, gewt, geb2d)


def _tc_mid(accp, gemb, xp, pw, pb2d, ng2d, nb2d, wn, bs, bd):
    dp = xp.shape[1]
    return pl.pallas_call(
        _tc_mid_kernel,
        grid=(N // BN,),
        in_specs=[
            pl.BlockSpec((NC, BN, HID), lambda i: (0, i, 0)),
            pl.BlockSpec((BN, HID), lambda i: (i, 0)),
            pl.BlockSpec((BN, dp), lambda i: (i, 0)),
            _full((HID, dp)), _full((1, HID)), _full((1, HID)), _full((1, HID)),
            _full((NH, HID)), _full((NH, HEADS)), _full((NH, HEADS)),
        ],
        out_specs=[
            pl.BlockSpec((BN, HID), lambda i: (i, 0)),
            pl.BlockSpec((BN, NH), lambda i: (i, 0)),
            pl.BlockSpec((BN, 16), lambda i: (i, 0)),
            pl.BlockSpec((BN, 16), lambda i: (i, 0)),
        ],
        out_shape=[
            jax.ShapeDtypeStruct((N, HID), jnp.float32),
            jax.ShapeDtypeStruct((N, NH), jnp.float32),
            jax.ShapeDtypeStruct((N, 16), jnp.float32),
            jax.ShapeDtypeStruct((N, 16), jnp.float32),
        ],
    )(accp, gemb, xp, pw, pb2d, ng2d, nb2d, wn, bs, bd)


def _tc_last(accp, gemb, xp, pw, pb2d, ng2d, nb2d):
    dp = xp.shape[1]
    return pl.pallas_call(
        _tc_last_kernel,
        grid=(N // BN,),
        in_specs=[
            pl.BlockSpec((NC, BN, HID), lambda i: (0, i, 0)),
            pl.BlockSpec((BN, HID), lambda i: (i, 0)),
            pl.BlockSpec((BN, dp), lambda i: (i, 0)),
            _full((HID, dp)), _full((1, HID)), _full((1, HID)), _full((1, HID)),
        ],
        out_specs=pl.BlockSpec((BN, HID), lambda i: (i, 0)),
        out_shape=jax.ShapeDtypeStruct((N, HID), jnp.float32),
    )(accp, gemb, xp, pw, pb2d, ng2d, nb2d)


def _tc_final(h3, c1w, c1b2d, c2w, c2b2d):
    return pl.pallas_call(
        _tc_final_kernel,
        in_specs=[_full((N, HID)), _full((HID // 2, HID)),
                  _full((1, HID // 2)), _full((1, HID // 2)), _full((1, 1))],
        out_specs=_full((1, 1)),
        out_shape=jax.ShapeDtypeStruct((1, 1), jnp.float32),
    )(h3, c1w, c1b2d, c2w, c2b2d)


def _block_diag(att):
    eye = jnp.eye(HEADS, dtype=jnp.float32)
    return jnp.einsum('hd,hg->hdg', att[0], eye).reshape(NH, HEADS)


def kernel(x, edge_index, gambling_scores, W1, as1, ad1, W2, as2, ad2, W3, as3, ad3, p1w, p1b, p2w, p2b, p3w, p3b, n1g, n1b, n2g, n2b, n3g, n3b, gew, geb, c1w, c1b, c2w, c2b):
    src = edge_index[0]
    dst = edge_index[1]
    gs = gambling_scores
    z64 = jnp.zeros((N, HID), jnp.float32)
    gs2d = gs[:, None]
    gewt = gew.T
    geb2d = geb[None, :]

    pass1 = _make_pass1()
    pass2 = _make_pass2()

    def layer_sparse(hf, ab, ba):
        denom_p, v = pass1(ab, ba, gs, src, dst)
        d16 = _tc_dcomb(denom_p.reshape(NW, N, HEADS))
        acc_p, = pass2(hf, v, d16, src, dst, z64)
        return acc_p

    h1, ab1, ba1, gemb = _tc_pre(x, W1, _block_diag(as1), _block_diag(ad1),
                                 gs2d, gewt, geb2d)
    acc1 = layer_sparse(h1, ab1, ba1)
    ho1, h2, ab2, ba2 = _tc_mid(acc1, gemb, x, p1w, p1b[None, :],
                                n1g[None, :], n1b[None, :], W2,
                                _block_diag(as2), _block_diag(ad2))
    acc2 = layer_sparse(h2, ab2, ba2)
    ho2, h3, ab3, ba3 = _tc_mid(acc2, gemb, ho1, p2w, p2b[None, :],
                                n2g[None, :], n2b[None, :], W3,
                                _block_diag(as3), _block_diag(ad3))
    acc3 = layer_sparse(h3, ab3, ba3)
    ho3 = _tc_last(acc3, gemb, ho2, p3w, p3b[None, :],
                   n3g[None, :], n3b[None, :])
    return _tc_final(ho3, c1w, c1b[None, :], c2w, c2b[None, :])


# speedup vs baseline: 40.6954x; 1.3520x over previous
import functools

import jax
import jax.numpy as jnp
from jax import lax
from jax.experimental import pallas as pl
from jax.experimental.pallas import tpu as pltpu
from jax.experimental.pallas import tpu_sc as plsc

N = 10000
E = 320000
HID = 64
HEADS = 8
NH = HEADS * HID
NEG_SLOPE = 0.2

NC = 2
NS = 16
NW = NC * NS
EW = E // NW
C1 = 80
C2 = 40
NCH1 = EW // C1
NCH2 = EW // C2
BN = 1000

_MESH = plsc.VectorSubcoreMesh(
    core_axis_name="c", subcore_axis_name="s", num_cores=NC, num_subcores=NS)


def _sc_pass1(ab_hbm, ba_hbm, gs_hbm, src_hbm, dst_hbm,
              denom_out, v_out,
              gs_v, denom_t,
              srcA, dstA, r1A, r2A, vbA, semIA, semGA,
              srcB, dstB, r1B, r2B, vbB, semIB, semGB):
    c_idx = lax.axis_index("c")
    s_idx = lax.axis_index("s")
    wid = s_idx * NC + c_idx
    iota = lax.iota(jnp.int32, 16)
    headmask = iota < 8

    pltpu.sync_copy(gs_hbm, gs_v)

    def zero(i, _):
        denom_t[pl.ds(i * 16, 16)] = jnp.zeros((16,), jnp.float32)
        return _

    lax.fori_loop(0, N * HEADS // 16, zero, None)

    def issue_idx(g, s_v, d_v, semI):
        e0 = wid * EW + g * C1
        pltpu.async_copy(src_hbm.at[pl.ds(e0, C1)], s_v, semI)
        pltpu.async_copy(dst_hbm.at[pl.ds(e0, C1)], d_v, semI)

    def wait_idx(s_v, d_v, semI):
        pltpu.make_async_copy(src_hbm.at[pl.ds(0, C1)], s_v, semI).wait()
        pltpu.make_async_copy(dst_hbm.at[pl.ds(0, C1)], d_v, semI).wait()

    def issue_g(s_v, d_v, r1, r2, semG):
        pltpu.async_copy(ab_hbm.at[d_v], r1, semG)
        pltpu.async_copy(ba_hbm.at[s_v], r2, semG)

    def wait_g(s_v, d_v, r1, r2, semG):
        pltpu.make_async_copy(ab_hbm.at[d_v], r1, semG).wait()
        pltpu.make_async_copy(ba_hbm.at[s_v], r2, semG).wait()

    def compute(g, s_v, d_v, r1, r2, vbuf):
        def edge(e, _):
            s = r1[e, :] + r2[e, :]
            u = jnp.exp(jnp.maximum(s, NEG_SLOPE * s))
            esplat = jnp.full((16,), e, jnp.int32)
            srcsplat = plsc.load_gather(s_v, [esplat])
            gsv = plsc.load_gather(gs_v, [srcsplat])
            vbuf[e, :] = u * (1.0 + gsv)
            dstsplat = plsc.load_gather(d_v, [esplat])
            plsc.addupdate_scatter(denom_t, [dstsplat * 8 + iota], u,
                                   mask=headmask)
            return _

        lax.fori_loop(0, C1, edge, None)
        e0 = wid * EW + g * C1
        pltpu.sync_copy(vbuf, v_out.at[pl.ds(e0, C1)])

    A = (srcA, dstA, r1A, r2A, vbA, semIA, semGA)
    B = (srcB, dstB, r1B, r2B, vbB, semIB, semGB)

    def _issue_idx(S, g):
        issue_idx(g, S[0], S[1], S[5])

    def _wait_idx(S):
        wait_idx(S[0], S[1], S[5])

    def _issue_g(S):
        issue_g(S[0], S[1], S[2], S[3], S[6])

    def _wait_g(S):
        wait_g(S[0], S[1], S[2], S[3], S[6])

    def _compute(S, g):
        compute(g, S[0], S[1], S[2], S[3], S[4])

    _issue_idx(A, 0)
    _wait_idx(A)
    _issue_g(A)

    def pair(p, _):
        g0 = 2 * p
        _issue_idx(B, g0 + 1)
        _wait_g(A)
        _wait_idx(B)
        _issue_g(B)
        _compute(A, g0)
        _issue_idx(A, g0 + 2)
        _wait_g(B)
        _wait_idx(A)
        _issue_g(A)
        _compute(B, g0 + 1)
        return _

    lax.fori_loop(0, (NCH1 - 1) // 2, pair, None)
    _wait_g(A)
    _compute(A, NCH1 - 1)
    pltpu.sync_copy(denom_t, denom_out.at[wid])


def _sc_pass2(hf_hbm, v_hbm, d16_hbm, src_hbm, dst_hbm, z64_hbm,
              acc_out,
              acc_sh,
              srcA, dstA, vbA, drA, hfA, cbA, msgA, semIA, semGA,
              srcB, dstB, vbB, drB, hfB, cbB, msgB, semIB, semGB):
    c_idx = lax.axis_index("c")
    s_idx = lax.axis_index("s")
    wid = s_idx * NC + c_idx
    iota = lax.iota(jnp.int32, 16)

    pltpu.sync_copy(z64_hbm.at[pl.ds(s_idx * 625, 625)],
                    acc_sh.at[pl.ds(s_idx * 625, 625)])
    plsc.subcore_barrier()

    def issue_idx(g, s_v, d_v, semI):
        e0 = wid * EW + g * C2
        pltpu.async_copy(src_hbm.at[pl.ds(e0, C2)], s_v, semI)
        pltpu.async_copy(dst_hbm.at[pl.ds(e0, C2)], d_v, semI)

    def wait_idx(s_v, d_v, semI):
        pltpu.make_async_copy(src_hbm.at[pl.ds(0, C2)], s_v, semI).wait()
        pltpu.make_async_copy(dst_hbm.at[pl.ds(0, C2)], d_v, semI).wait()

    def issue_g(g, s_v, d_v, vbuf, dr, hfb, semG):
        e0 = wid * EW + g * C2
        pltpu.async_copy(hf_hbm.at[s_v], hfb, semG)
        pltpu.async_copy(v_hbm.at[pl.ds(e0, C2)], vbuf, semG)
        pltpu.async_copy(d16_hbm.at[d_v], dr, semG)

    def wait_g(s_v, d_v, vbuf, dr, hfb, semG):
        pltpu.make_async_copy(hf_hbm.at[s_v], hfb, semG).wait()
        pltpu.make_async_copy(v_hbm.at[pl.ds(0, C2)], vbuf, semG).wait()
        pltpu.make_async_copy(d16_hbm.at[d_v], dr, semG).wait()

    def compute(s_v, d_v, vbuf, dr, hfb, cbuf, msg):
        def coeff(e, _):
            vv = vbuf[e, :]
            dv = dr[e, :]
            c = vv / ((dv + 1e-16) * float(HEADS))
            cbuf[pl.ds(e * 16, 16)] = c
            return _

        lax.fori_loop(0, C2, coeff, None)

        def edge(e, _):
            m = [jnp.zeros((16,), jnp.float32) for _ in range(4)]
            for h in range(HEADS):
                cb = plsc.load_gather(
                    cbuf, [jnp.full((16,), e * 16 + h, jnp.int32)])
                for j in range(4):
                    hv = hfb[e, pl.ds(h * HID + j * 16, 16)]
                    m[j] = m[j] + cb * hv
            for j in range(4):
                msg[e, pl.ds(j * 16, 16)] = m[j]
            return _

        lax.fori_loop(0, C2, edge, None)
        pltpu.sync_copy(msg, acc_sh.at[d_v], add=True)

    A = (srcA, dstA, vbA, drA, hfA, cbA, msgA, semIA, semGA)
    B = (srcB, dstB, vbB, drB, hfB, cbB, msgB, semIB, semGB)

    def _issue_idx(S, g):
        issue_idx(g, S[0], S[1], S[7])

    def _wait_idx(S):
        wait_idx(S[0], S[1], S[7])

    def _issue_g(S, g):
        issue_g(g, S[0], S[1], S[2], S[3], S[4], S[8])

    def _wait_g(S):
        wait_g(S[0], S[1], S[2], S[3], S[4], S[8])

    def _compute(S):
        compute(S[0], S[1], S[2], S[3], S[4], S[5], S[6])

    _issue_idx(A, 0)
    _wait_idx(A)
    _issue_g(A, 0)

    def pair(p, _):
        g0 = 2 * p
        _issue_idx(B, g0 + 1)
        _wait_g(A)
        _wait_idx(B)
        _issue_g(B, g0 + 1)
        _compute(A)
        _issue_idx(A, g0 + 2)
        _wait_g(B)
        _wait_idx(A)
        _issue_g(A, g0 + 2)
        _compute(B)
        return _

    lax.fori_loop(0, (NCH2 - 1) // 2, pair, None)
    if NCH2 % 2 == 0:
        _issue_idx(B, NCH2 - 1)
        _wait_g(A)
        _wait_idx(B)
        _issue_g(B, NCH2 - 1)
        _compute(A)
        _wait_g(B)
        _compute(B)
    else:
        _wait_g(A)
        _compute(A)
    plsc.subcore_barrier()
    pltpu.sync_copy(acc_sh.at[pl.ds(s_idx * 625, 625)],
                    acc_out.at[c_idx, pl.ds(s_idx * 625, 625)])


def _make_pass1():
    return functools.partial(
        pl.kernel,
        out_type=[jax.ShapeDtypeStruct((NW, N * HEADS), jnp.float32),
                  jax.ShapeDtypeStruct((E, 16), jnp.float32)],
        mesh=_MESH,
        scratch_types=[
            pltpu.VMEM((N,), jnp.float32),
            pltpu.VMEM((N * HEADS,), jnp.float32),
        ] + 2 * [
            pltpu.VMEM((C1,), jnp.int32),
            pltpu.VMEM((C1,), jnp.int32),
            pltpu.VMEM((C1, 16), jnp.float32),
            pltpu.VMEM((C1, 16), jnp.float32),
            pltpu.VMEM((C1, 16), jnp.float32),
            pltpu.SemaphoreType.DMA,
            pltpu.SemaphoreType.DMA,
        ],
        compiler_params=pltpu.CompilerParams(needs_layout_passes=False, use_tc_tiling_on_sc=False),
    )(_sc_pass1)


def _make_pass2():
    return functools.partial(
        pl.kernel,
        out_type=[jax.ShapeDtypeStruct((NC, N, HID), jnp.float32)],
        mesh=_MESH,
        scratch_types=[
            pltpu.VMEM_SHARED((N, HID), jnp.float32),
        ] + 2 * [
            pltpu.VMEM((C2,), jnp.int32),
            pltpu.VMEM((C2,), jnp.int32),
            pltpu.VMEM((C2, 16), jnp.float32),
            pltpu.VMEM((C2, 16), jnp.float32),
            pltpu.VMEM((C2, NH), jnp.float32),
            pltpu.VMEM((C2 * 16,), jnp.float32),
            pltpu.VMEM((C2, HID), jnp.float32),
            pltpu.SemaphoreType.DMA,
            pltpu.SemaphoreType.DMA,
        ],
        compiler_params=pltpu.CompilerParams(needs_layout_passes=False, use_tc_tiling_on_sc=False),
    )(_sc_pass2)


def _tc_pre_kernel(x_ref, w1_ref, bs_ref, bd_ref, gs_ref, gewt_ref, geb_ref,
                   h_ref, ab_ref, ba_ref, gemb_ref):
    h = lax.dot_general(x_ref[...], w1_ref[...], (((1,), (1,)), ((), ())),
                        preferred_element_type=jnp.float32)
    h_ref[...] = h
    a1 = jnp.dot(h, bs_ref[...], preferred_element_type=jnp.float32)
    a2 = jnp.dot(h, bd_ref[...], preferred_element_type=jnp.float32)
    ab_ref[...] = jnp.concatenate([a1, a2], axis=-1)
    ba_ref[...] = jnp.concatenate([a2, a1], axis=-1)
    gemb_ref[...] = jnp.dot(gs_ref[...], gewt_ref[...],
                            preferred_element_type=jnp.float32) + geb_ref[...]


def _tc_dcomb_kernel(dp_ref, d16_ref):
    d = jnp.sum(dp_ref[...], axis=0)
    d16_ref[...] = jnp.concatenate([d, jnp.ones_like(d)], axis=-1)


def _tc_dcomb(denom_p):
    return pl.pallas_call(
        _tc_dcomb_kernel,
        grid=(N // BN,),
        in_specs=[pl.BlockSpec((NW, BN, HEADS), lambda i: (0, i, 0))],
        out_specs=pl.BlockSpec((BN, 16), lambda i: (i, 0)),
        out_shape=jax.ShapeDtypeStruct((N, 16), jnp.float32),
    )(denom_p)


def _tc_mid_kernel(acc_ref, gemb_ref, xp_ref, pw_ref, pb_ref, ng_ref, nb_ref,
                   wn_ref, bs_ref, bd_ref,
                   ho_ref, hn_ref, ab_ref, ba_ref):
    s = acc_ref[0] + acc_ref[1] + gemb_ref[...]
    mu = jnp.mean(s, axis=-1, keepdims=True)
    var = jnp.mean((s - mu) ** 2, axis=-1, keepdims=True)
    ln = (s - mu) * lax.rsqrt(var + 1e-5) * ng_ref[...] + nb_ref[...]
    ident = lax.dot_general(xp_ref[...], pw_ref[...], (((1,), (1,)), ((), ())),
                            preferred_element_type=jnp.float32) + pb_ref[...]
    ho = jnp.maximum(ln + ident, 0.0)
    ho_ref[...] = ho
    hn = lax.dot_general(ho, wn_ref[...], (((1,), (1,)), ((), ())),
                         preferred_element_type=jnp.float32)
    hn_ref[...] = hn
    a1 = jnp.dot(hn, bs_ref[...], preferred_element_type=jnp.float32)
    a2 = jnp.dot(hn, bd_ref[...], preferred_element_type=jnp.float32)
    ab_ref[...] = jnp.concatenate([a1, a2], axis=-1)
    ba_ref[...] = jnp.concatenate([a2, a1], axis=-1)


def _tc_last_kernel(acc_ref, gemb_ref, xp_ref, pw_ref, pb_ref, ng_ref, nb_ref,
                    ho_ref):
    s = acc_ref[0] + acc_ref[1] + gemb_ref[...]
    mu = jnp.mean(s, axis=-1, keepdims=True)
    var = jnp.mean((s - mu) ** 2, axis=-1, keepdims=True)
    ln = (s - mu) * lax.rsqrt(var + 1e-5) * ng_ref[...] + nb_ref[...]
    ident = lax.dot_general(xp_ref[...], pw_ref[...], (((1,), (1,)), ((), ())),
                            preferred_element_type=jnp.float32) + pb_ref[...]
    ho_ref[...] = jnp.maximum(ln + ident, 0.0)


def _tc_final_kernel(h_ref, c1w_ref, c1b_ref, c2w_ref, c2b_ref, o_ref):
    pooled = jnp.mean(h_ref[...], axis=0, keepdims=True)
    z = lax.dot_general(pooled, c1w_ref[...], (((1,), (1,)), ((), ())),
                        preferred_element_type=jnp.float32) + c1b_ref[...]
    z = jnp.maximum(z, 0.0)
    z = jnp.sum(z * c2w_ref[...], axis=-1, keepdims=True) + c2b_ref[...]
    o_ref[...] = 1.0 / (1.0 + jnp.exp(-z))


def _full(shape):
    return pl.BlockSpec(shape, lambda *_: tuple(0 for _ in shape))


def _tc_pre(x, w1, bs, bd, gs2d, gewt, geb2d):
    din = x.shape[1]
    return pl.pallas_call(
        _tc_pre_kernel,
        grid=(N // BN,),
        in_specs=[
            pl.BlockSpec((BN, din), lambda i: (i, 0)),
            _full((NH, din)), _full((NH, HEADS)), _full((NH, HEADS)),
            pl.BlockSpec((BN, 1), lambda i: (i, 0)),
            _full((1, HID)), _full((1, HID)),
        ],
        out_specs=[
            pl.BlockSpec((BN, NH), lambda i: (i, 0)),
            pl.BlockSpec((BN, 16), lambda i: (i, 0)),
            pl.BlockSpec((BN, 16), lambda i: (i, 0)),
            pl.BlockSpec((BN, HID), lambda i: (i, 0)),
        ],
        out_shape=[
            jax.ShapeDtypeStruct((N, NH), jnp.float32),
            jax.ShapeDtypeStruct((N, 16), jnp.float32),
            jax.ShapeDtypeStruct((N, 16), jnp.float32),
            jax.ShapeDtypeStruct((N, HID), jnp.float32),
        ],
    )(x, w1, bs, bd, gs2d, gewt, geb2d)


def _tc_mid(accp, gemb, xp, pw, pb2d, ng2d, nb2d, wn, bs, bd):
    dp = xp.shape[1]
    return pl.pallas_call(
        _tc_mid_kernel,
        grid=(N // BN,),
        in_specs=[
            pl.BlockSpec((NC, BN, HID), lambda i: (0, i, 0)),
            pl.BlockSpec((BN, HID), lambda i: (i, 0)),
            pl.BlockSpec((BN, dp), lambda i: (i, 0)),
            _full((HID, dp)), _full((1, HID)), _full((1, HID)), _full((1, HID)),
            _full((NH, HID)), _full((NH, HEADS)), _full((NH, HEADS)),
        ],
        out_specs=[
            pl.BlockSpec((BN, HID), lambda i: (i, 0)),
            pl.BlockSpec((BN, NH), lambda i: (i, 0)),
            pl.BlockSpec((BN, 16), lambda i: (i, 0)),
            pl.BlockSpec((BN, 16), lambda i: (i, 0)),
        ],
        out_shape=[
            jax.ShapeDtypeStruct((N, HID), jnp.float32),
            jax.ShapeDtypeStruct((N, NH), jnp.float32),
            jax.ShapeDtypeStruct((N, 16), jnp.float32),
            jax.ShapeDtypeStruct((N, 16), jnp.float32),
        ],
    )(accp, gemb, xp, pw, pb2d, ng2d, nb2d, wn, bs, bd)


def _tc_last(accp, gemb, xp, pw, pb2d, ng2d, nb2d):
    dp = xp.shape[1]
    return pl.pallas_call(
        _tc_last_kernel,
        grid=(N // BN,),
        in_specs=[
            pl.BlockSpec((NC, BN, HID), lambda i: (0, i, 0)),
            pl.BlockSpec((BN, HID), lambda i: (i, 0)),
            pl.BlockSpec((BN, dp), lambda i: (i, 0)),
            _full((HID, dp)), _full((1, HID)), _full((1, HID)), _full((1, HID)),
        ],
        out_specs=pl.BlockSpec((BN, HID), lambda i: (i, 0)),
        out_shape=jax.ShapeDtypeStruct((N, HID), jnp.float32),
    )(accp, gemb, xp, pw, pb2d, ng2d, nb2d)


def _tc_final(h3, c1w, c1b2d, c2w, c2b2d):
    return pl.pallas_call(
        _tc_final_kernel,
        in_specs=[_full((N, HID)), _full((HID // 2, HID)),
                  _full((1, HID // 2)), _full((1, HID // 2)), _full((1, 1))],
        out_specs=_full((1, 1)),
        out_shape=jax.ShapeDtypeStruct((1, 1), jnp.float32),
    )(h3, c1w, c1b2d, c2w, c2b2d)


def _block_diag(att):
    eye = jnp.eye(HEADS, dtype=jnp.float32)
    return jnp.einsum('hd,hg->hdg', att[0], eye).reshape(NH, HEADS)


def kernel(x, edge_index, gambling_scores, W1, as1, ad1, W2, as2, ad2, W3, as3, ad3, p1w, p1b, p2w, p2b, p3w, p3b, n1g, n1b, n2g, n2b, n3g, n3b, gew, geb, c1w, c1b, c2w, c2b):
    src = edge_index[0]
    dst = edge_index[1]
    gs = gambling_scores
    z64 = jnp.zeros((N, HID), jnp.float32)
    gs2d = gs[:, None]
    gewt = gew.T
    geb2d = geb[None, :]

    pass1 = _make_pass1()
    pass2 = _make_pass2()

    def layer_sparse(hf, ab, ba):
        denom_p, v = pass1(ab, ba, gs, src, dst)
        d16 = _tc_dcomb(denom_p.reshape(NW, N, HEADS))
        acc_p, = pass2(hf, v, d16, src, dst, z64)
        return acc_p

    h1, ab1, ba1, gemb = _tc_pre(x, W1, _block_diag(as1), _block_diag(ad1),
                                 gs2d, gewt, geb2d)
    acc1 = layer_sparse(h1, ab1, ba1)
    ho1, h2, ab2, ba2 = _tc_mid(acc1, gemb, x, p1w, p1b[None, :],
                                n1g[None, :], n1b[None, :], W2,
                                _block_diag(as2), _block_diag(ad2))
    acc2 = layer_sparse(h2, ab2, ba2)
    ho2, h3, ab3, ba3 = _tc_mid(acc2, gemb, ho1, p2w, p2b[None, :],
                                n2g[None, :], n2b[None, :], W3,
                                _block_diag(as3), _block_diag(ad3))
    acc3 = layer_sparse(h3, ab3, ba3)
    ho3 = _tc_last(acc3, gemb, ho2, p3w, p3b[None, :],
                   n3g[None, :], n3b[None, :])
    return _tc_final(ho3, c1w, c1b[None, :], c2w, c2b[None, :])


# speedup vs baseline: 54.4196x; 1.3372x over previous
import functools

import jax
import jax.numpy as jnp
from jax import lax
from jax.experimental import pallas as pl
from jax.experimental.pallas import tpu as pltpu
from jax.experimental.pallas import tpu_sc as plsc

N = 10000
E = 320000
HID = 64
HEADS = 8
NH = HEADS * HID
NEG_SLOPE = 0.2

NC = 2
NS = 16
NW = NC * NS
EW = E // NW
C1 = 80
C2 = 40
NCH1 = EW // C1
NCH2 = EW // C2
BN = 1000

_MESH = plsc.VectorSubcoreMesh(
    core_axis_name="c", subcore_axis_name="s", num_cores=NC, num_subcores=NS)


def _sc_pass1(ab_hbm, ba_hbm, gs_hbm, src_hbm, dst_hbm,
              denom_out, v_out,
              gs_v, denom_t,
              srcA, dstA, r1A, r2A, vbA, semIA, semGA,
              srcB, dstB, r1B, r2B, vbB, semIB, semGB):
    c_idx = lax.axis_index("c")
    s_idx = lax.axis_index("s")
    wid = s_idx * NC + c_idx
    iota = lax.iota(jnp.int32, 16)
    headmask = iota < 8

    pltpu.sync_copy(gs_hbm, gs_v)

    @plsc.parallel_loop(0, N * HEADS // 16, unroll=8)
    def zero(i):
        denom_t[pl.ds(i * 16, 16)] = jnp.zeros((16,), jnp.float32)

    def issue_idx(g, s_v, d_v, semI):
        e0 = wid * EW + g * C1
        pltpu.async_copy(src_hbm.at[pl.ds(e0, C1)], s_v, semI)
        pltpu.async_copy(dst_hbm.at[pl.ds(e0, C1)], d_v, semI)

    def wait_idx(s_v, d_v, semI):
        pltpu.make_async_copy(src_hbm.at[pl.ds(0, C1)], s_v, semI).wait()
        pltpu.make_async_copy(dst_hbm.at[pl.ds(0, C1)], d_v, semI).wait()

    def issue_g(s_v, d_v, r1, r2, semG):
        pltpu.async_copy(ab_hbm.at[d_v], r1, semG)
        pltpu.async_copy(ba_hbm.at[s_v], r2, semG)

    def wait_g(s_v, d_v, r1, r2, semG):
        pltpu.make_async_copy(ab_hbm.at[d_v], r1, semG).wait()
        pltpu.make_async_copy(ba_hbm.at[s_v], r2, semG).wait()

    def compute(g, s_v, d_v, r1, r2, vbuf):
        @plsc.parallel_loop(0, C1, unroll=4)
        def edge(e):
            s = r1[e, :] + r2[e, :]
            u = jnp.exp(jnp.maximum(s, NEG_SLOPE * s))
            esplat = jnp.full((16,), e, jnp.int32)
            srcsplat = plsc.load_gather(s_v, [esplat])
            gsv = plsc.load_gather(gs_v, [srcsplat])
            vbuf[e, :] = u * (1.0 + gsv)
            dstsplat = plsc.load_gather(d_v, [esplat])
            plsc.addupdate_scatter(denom_t, [dstsplat * 8 + iota], u,
                                   mask=headmask)

        e0 = wid * EW + g * C1
        pltpu.sync_copy(vbuf, v_out.at[pl.ds(e0, C1)])

    A = (srcA, dstA, r1A, r2A, vbA, semIA, semGA)
    B = (srcB, dstB, r1B, r2B, vbB, semIB, semGB)

    def _issue_idx(S, g):
        issue_idx(g, S[0], S[1], S[5])

    def _wait_idx(S):
        wait_idx(S[0], S[1], S[5])

    def _issue_g(S):
        issue_g(S[0], S[1], S[2], S[3], S[6])

    def _wait_g(S):
        wait_g(S[0], S[1], S[2], S[3], S[6])

    def _compute(S, g):
        compute(g, S[0], S[1], S[2], S[3], S[4])

    _issue_idx(A, 0)
    _wait_idx(A)
    _issue_g(A)

    def pair(p, _):
        g0 = 2 * p
        _issue_idx(B, g0 + 1)
        _wait_g(A)
        _wait_idx(B)
        _issue_g(B)
        _compute(A, g0)
        _issue_idx(A, g0 + 2)
        _wait_g(B)
        _wait_idx(A)
        _issue_g(A)
        _compute(B, g0 + 1)
        return _

    lax.fori_loop(0, (NCH1 - 1) // 2, pair, None)
    _wait_g(A)
    _compute(A, NCH1 - 1)
    pltpu.sync_copy(denom_t, denom_out.at[wid])


def _sc_pass2(hf_hbm, v_hbm, d16_hbm, src_hbm, dst_hbm, z64_hbm,
              acc_out,
              acc_sh,
              srcA, dstA, vbA, drA, hfA, cbA, msgA, semIA, semGA,
              srcB, dstB, vbB, drB, hfB, cbB, msgB, semIB, semGB):
    c_idx = lax.axis_index("c")
    s_idx = lax.axis_index("s")
    wid = s_idx * NC + c_idx
    iota = lax.iota(jnp.int32, 16)

    pltpu.sync_copy(z64_hbm.at[pl.ds(s_idx * 625, 625)],
                    acc_sh.at[pl.ds(s_idx * 625, 625)])
    plsc.subcore_barrier()

    def issue_idx(g, s_v, d_v, semI):
        e0 = wid * EW + g * C2
        pltpu.async_copy(src_hbm.at[pl.ds(e0, C2)], s_v, semI)
        pltpu.async_copy(dst_hbm.at[pl.ds(e0, C2)], d_v, semI)

    def wait_idx(s_v, d_v, semI):
        pltpu.make_async_copy(src_hbm.at[pl.ds(0, C2)], s_v, semI).wait()
        pltpu.make_async_copy(dst_hbm.at[pl.ds(0, C2)], d_v, semI).wait()

    def issue_g(g, s_v, d_v, vbuf, dr, hfb, semG):
        e0 = wid * EW + g * C2
        pltpu.async_copy(hf_hbm.at[s_v], hfb, semG)
        pltpu.async_copy(v_hbm.at[pl.ds(e0, C2)], vbuf, semG)
        pltpu.async_copy(d16_hbm.at[d_v], dr, semG)

    def wait_g(s_v, d_v, vbuf, dr, hfb, semG):
        pltpu.make_async_copy(hf_hbm.at[s_v], hfb, semG).wait()
        pltpu.make_async_copy(v_hbm.at[pl.ds(0, C2)], vbuf, semG).wait()
        pltpu.make_async_copy(d16_hbm.at[d_v], dr, semG).wait()

    def compute(s_v, d_v, vbuf, dr, hfb, cbuf, msg):
        @plsc.parallel_loop(0, C2, unroll=4)
        def coeff(e):
            vv = vbuf[e, :]
            dv = dr[e, :]
            c = vv / ((dv + 1e-16) * float(HEADS))
            cbuf[pl.ds(e * 16, 16)] = c

        @plsc.parallel_loop(0, C2, unroll=2)
        def edge(e):
            m = [jnp.zeros((16,), jnp.float32) for _ in range(4)]
            for h in range(HEADS):
                cb = plsc.load_gather(
                    cbuf, [jnp.full((16,), e * 16 + h, jnp.int32)])
                for j in range(4):
                    hv = hfb[e, pl.ds(h * HID + j * 16, 16)]
                    m[j] = m[j] + cb * hv
            for j in range(4):
                msg[e, pl.ds(j * 16, 16)] = m[j]

        pltpu.sync_copy(msg, acc_sh.at[d_v], add=True)

    A = (srcA, dstA, vbA, drA, hfA, cbA, msgA, semIA, semGA)
    B = (srcB, dstB, vbB, drB, hfB, cbB, msgB, semIB, semGB)

    def _issue_idx(S, g):
        issue_idx(g, S[0], S[1], S[7])

    def _wait_idx(S):
        wait_idx(S[0], S[1], S[7])

    def _issue_g(S, g):
        issue_g(g, S[0], S[1], S[2], S[3], S[4], S[8])

    def _wait_g(S):
        wait_g(S[0], S[1], S[2], S[3], S[4], S[8])

    def _compute(S):
        compute(S[0], S[1], S[2], S[3], S[4], S[5], S[6])

    _issue_idx(A, 0)
    _wait_idx(A)
    _issue_g(A, 0)

    def pair(p, _):
        g0 = 2 * p
        _issue_idx(B, g0 + 1)
        _wait_g(A)
        _wait_idx(B)
        _issue_g(B, g0 + 1)
        _compute(A)
        _issue_idx(A, g0 + 2)
        _wait_g(B)
        _wait_idx(A)
        _issue_g(A, g0 + 2)
        _compute(B)
        return _

    lax.fori_loop(0, (NCH2 - 1) // 2, pair, None)
    if NCH2 % 2 == 0:
        _issue_idx(B, NCH2 - 1)
        _wait_g(A)
        _wait_idx(B)
        _issue_g(B, NCH2 - 1)
        _compute(A)
        _wait_g(B)
        _compute(B)
    else:
        _wait_g(A)
        _compute(A)
    plsc.subcore_barrier()
    pltpu.sync_copy(acc_sh.at[pl.ds(s_idx * 625, 625)],
                    acc_out.at[c_idx, pl.ds(s_idx * 625, 625)])


def _make_pass1():
    return functools.partial(
        pl.kernel,
        out_type=[jax.ShapeDtypeStruct((NW, N * HEADS), jnp.float32),
                  jax.ShapeDtypeStruct((E, 16), jnp.float32)],
        mesh=_MESH,
        scratch_types=[
            pltpu.VMEM((N,), jnp.float32),
            pltpu.VMEM((N * HEADS,), jnp.float32),
        ] + 2 * [
            pltpu.VMEM((C1,), jnp.int32),
            pltpu.VMEM((C1,), jnp.int32),
            pltpu.VMEM((C1, 16), jnp.float32),
            pltpu.VMEM((C1, 16), jnp.float32),
            pltpu.VMEM((C1, 16), jnp.float32),
            pltpu.SemaphoreType.DMA,
            pltpu.SemaphoreType.DMA,
        ],
        compiler_params=pltpu.CompilerParams(needs_layout_passes=False, use_tc_tiling_on_sc=False),
    )(_sc_pass1)


def _make_pass2():
    return functools.partial(
        pl.kernel,
        out_type=[jax.ShapeDtypeStruct((NC, N, HID), jnp.float32)],
        mesh=_MESH,
        scratch_types=[
            pltpu.VMEM_SHARED((N, HID), jnp.float32),
        ] + 2 * [
            pltpu.VMEM((C2,), jnp.int32),
            pltpu.VMEM((C2,), jnp.int32),
            pltpu.VMEM((C2, 16), jnp.float32),
            pltpu.VMEM((C2, 16), jnp.float32),
            pltpu.VMEM((C2, NH), jnp.float32),
            pltpu.VMEM((C2 * 16,), jnp.float32),
            pltpu.VMEM((C2, HID), jnp.float32),
            pltpu.SemaphoreType.DMA,
            pltpu.SemaphoreType.DMA,
        ],
        compiler_params=pltpu.CompilerParams(needs_layout_passes=False, use_tc_tiling_on_sc=False),
    )(_sc_pass2)


def _tc_pre_kernel(x_ref, w1_ref, bs_ref, bd_ref, gs_ref, gewt_ref, geb_ref,
                   h_ref, ab_ref, ba_ref, gemb_ref):
    h = lax.dot_general(x_ref[...], w1_ref[...], (((1,), (1,)), ((), ())),
                        preferred_element_type=jnp.float32)
    h_ref[...] = h
    a1 = jnp.dot(h, bs_ref[...], preferred_element_type=jnp.float32)
    a2 = jnp.dot(h, bd_ref[...], preferred_element_type=jnp.float32)
    ab_ref[...] = jnp.concatenate([a1, a2], axis=-1)
    ba_ref[...] = jnp.concatenate([a2, a1], axis=-1)
    gemb_ref[...] = jnp.dot(gs_ref[...], gewt_ref[...],
                            preferred_element_type=jnp.float32) + geb_ref[...]


def _tc_dcomb_kernel(dp_ref, d16_ref):
    d = jnp.sum(dp_ref[...], axis=0)
    d16_ref[...] = jnp.concatenate([d, jnp.ones_like(d)], axis=-1)


def _tc_dcomb(denom_p):
    return pl.pallas_call(
        _tc_dcomb_kernel,
        grid=(N // BN,),
        in_specs=[pl.BlockSpec((NW, BN, HEADS), lambda i: (0, i, 0))],
        out_specs=pl.BlockSpec((BN, 16), lambda i: (i, 0)),
        out_shape=jax.ShapeDtypeStruct((N, 16), jnp.float32),
    )(denom_p)


def _tc_mid_kernel(acc_ref, gemb_ref, xp_ref, pw_ref, pb_ref, ng_ref, nb_ref,
                   wn_ref, bs_ref, bd_ref,
                   ho_ref, hn_ref, ab_ref, ba_ref):
    s = acc_ref[0] + acc_ref[1] + gemb_ref[...]
    mu = jnp.mean(s, axis=-1, keepdims=True)
    var = jnp.mean((s - mu) ** 2, axis=-1, keepdims=True)
    ln = (s - mu) * lax.rsqrt(var + 1e-5) * ng_ref[...] + nb_ref[...]
    ident = lax.dot_general(xp_ref[...], pw_ref[...], (((1,), (1,)), ((), ())),
                            preferred_element_type=jnp.float32) + pb_ref[...]
    ho = jnp.maximum(ln + ident, 0.0)
    ho_ref[...] = ho
    hn = lax.dot_general(ho, wn_ref[...], (((1,), (1,)), ((), ())),
                         preferred_element_type=jnp.float32)
    hn_ref[...] = hn
    a1 = jnp.dot(hn, bs_ref[...], preferred_element_type=jnp.float32)
    a2 = jnp.dot(hn, bd_ref[...], preferred_element_type=jnp.float32)
    ab_ref[...] = jnp.concatenate([a1, a2], axis=-1)
    ba_ref[...] = jnp.concatenate([a2, a1], axis=-1)


def _tc_last_kernel(acc_ref, gemb_ref, xp_ref, pw_ref, pb_ref, ng_ref, nb_ref,
                    ho_ref):
    s = acc_ref[0] + acc_ref[1] + gemb_ref[...]
    mu = jnp.mean(s, axis=-1, keepdims=True)
    var = jnp.mean((s - mu) ** 2, axis=-1, keepdims=True)
    ln = (s - mu) * lax.rsqrt(var + 1e-5) * ng_ref[...] + nb_ref[...]
    ident = lax.dot_general(xp_ref[...], pw_ref[...], (((1,), (1,)), ((), ())),
                            preferred_element_type=jnp.float32) + pb_ref[...]
    ho_ref[...] = jnp.maximum(ln + ident, 0.0)


def _tc_final_kernel(h_ref, c1w_ref, c1b_ref, c2w_ref, c2b_ref, o_ref):
    pooled = jnp.mean(h_ref[...], axis=0, keepdims=True)
    z = lax.dot_general(pooled, c1w_ref[...], (((1,), (1,)), ((), ())),
                        preferred_element_type=jnp.float32) + c1b_ref[...]
    z = jnp.maximum(z, 0.0)
    z = jnp.sum(z * c2w_ref[...], axis=-1, keepdims=True) + c2b_ref[...]
    o_ref[...] = 1.0 / (1.0 + jnp.exp(-z))


def _full(shape):
    return pl.BlockSpec(shape, lambda *_: tuple(0 for _ in shape))


def _tc_pre(x, w1, bs, bd, gs2d, gewt, geb2d):
    din = x.shape[1]
    return pl.pallas_call(
        _tc_pre_kernel,
        grid=(N // BN,),
        in_specs=[
            pl.BlockSpec((BN, din), lambda i: (i, 0)),
            _full((NH, din)), _full((NH, HEADS)), _full((NH, HEADS)),
            pl.BlockSpec((BN, 1), lambda i: (i, 0)),
            _full((1, HID)), _full((1, HID)),
        ],
        out_specs=[
            pl.BlockSpec((BN, NH), lambda i: (i, 0)),
            pl.BlockSpec((BN, 16), lambda i: (i, 0)),
            pl.BlockSpec((BN, 16), lambda i: (i, 0)),
            pl.BlockSpec((BN, HID), lambda i: (i, 0)),
        ],
        out_shape=[
            jax.ShapeDtypeStruct((N, NH), jnp.float32),
            jax.ShapeDtypeStruct((N, 16), jnp.float32),
            jax.ShapeDtypeStruct((N, 16), jnp.float32),
            jax.ShapeDtypeStruct((N, HID), jnp.float32),
        ],
    )(x, w1, bs, bd, gs2d, gewt, geb2d)


def _tc_mid(accp, gemb, xp, pw, pb2d, ng2d, nb2d, wn, bs, bd):
    dp = xp.shape[1]
    return pl.pallas_call(
        _tc_mid_kernel,
        grid=(N // BN,),
        in_specs=[
            pl.BlockSpec((NC, BN, HID), lambda i: (0, i, 0)),
            pl.BlockSpec((BN, HID), lambda i: (i, 0)),
            pl.BlockSpec((BN, dp), lambda i: (i, 0)),
            _full((HID, dp)), _full((1, HID)), _full((1, HID)), _full((1, HID)),
            _full((NH, HID)), _full((NH, HEADS)), _full((NH, HEADS)),
        ],
        out_specs=[
            pl.BlockSpec((BN, HID), lambda i: (i, 0)),
            pl.BlockSpec((BN, NH), lambda i: (i, 0)),
            pl.BlockSpec((BN, 16), lambda i: (i, 0)),
            pl.BlockSpec((BN, 16), lambda i: (i, 0)),
        ],
        out_shape=[
            jax.ShapeDtypeStruct((N, HID), jnp.float32),
            jax.ShapeDtypeStruct((N, NH), jnp.float32),
            jax.ShapeDtypeStruct((N, 16), jnp.float32),
            jax.ShapeDtypeStruct((N, 16), jnp.float32),
        ],
    )(accp, gemb, xp, pw, pb2d, ng2d, nb2d, wn, bs, bd)


def _tc_last(accp, gemb, xp, pw, pb2d, ng2d, nb2d):
    dp = xp.shape[1]
    return pl.pallas_call(
        _tc_last_kernel,
        grid=(N // BN,),
        in_specs=[
            pl.BlockSpec((NC, BN, HID), lambda i: (0, i, 0)),
            pl.BlockSpec((BN, HID), lambda i: (i, 0)),
            pl.BlockSpec((BN, dp), lambda i: (i, 0)),
            _full((HID, dp)), _full((1, HID)), _full((1, HID)), _full((1, HID)),
        ],
        out_specs=pl.BlockSpec((BN, HID), lambda i: (i, 0)),
        out_shape=jax.ShapeDtypeStruct((N, HID), jnp.float32),
    )(accp, gemb, xp, pw, pb2d, ng2d, nb2d)


def _tc_final(h3, c1w, c1b2d, c2w, c2b2d):
    return pl.pallas_call(
        _tc_final_kernel,
        in_specs=[_full((N, HID)), _full((HID // 2, HID)),
                  _full((1, HID // 2)), _full((1, HID // 2)), _full((1, 1))],
        out_specs=_full((1, 1)),
        out_shape=jax.ShapeDtypeStruct((1, 1), jnp.float32),
    )(h3, c1w, c1b2d, c2w, c2b2d)


def _block_diag(att):
    eye = jnp.eye(HEADS, dtype=jnp.float32)
    return jnp.einsum('hd,hg->hdg', att[0], eye).reshape(NH, HEADS)


def kernel(x, edge_index, gambling_scores, W1, as1, ad1, W2, as2, ad2, W3, as3, ad3, p1w, p1b, p2w, p2b, p3w, p3b, n1g, n1b, n2g, n2b, n3g, n3b, gew, geb, c1w, c1b, c2w, c2b):
    src = edge_index[0]
    dst = edge_index[1]
    gs = gambling_scores
    z64 = jnp.zeros((N, HID), jnp.float32)
    gs2d = gs[:, None]
    gewt = gew.T
    geb2d = geb[None, :]

    pass1 = _make_pass1()
    pass2 = _make_pass2()

    def layer_sparse(hf, ab, ba):
        denom_p, v = pass1(ab, ba, gs, src, dst)
        d16 = _tc_dcomb(denom_p.reshape(NW, N, HEADS))
        acc_p, = pass2(hf, v, d16, src, dst, z64)
        return acc_p

    h1, ab1, ba1, gemb = _tc_pre(x, W1, _block_diag(as1), _block_diag(ad1),
                                 gs2d, gewt, geb2d)
    acc1 = layer_sparse(h1, ab1, ba1)
    ho1, h2, ab2, ba2 = _tc_mid(acc1, gemb, x, p1w, p1b[None, :],
                                n1g[None, :], n1b[None, :], W2,
                                _block_diag(as2), _block_diag(ad2))
    acc2 = layer_sparse(h2, ab2, ba2)
    ho2, h3, ab3, ba3 = _tc_mid(acc2, gemb, ho1, p2w, p2b[None, :],
                                n2g[None, :], n2b[None, :], W3,
                                _block_diag(as3), _block_diag(ad3))
    acc3 = layer_sparse(h3, ab3, ba3)
    ho3 = _tc_last(acc3, gemb, ho2, p3w, p3b[None, :],
                   n3g[None, :], n3b[None, :])
    return _tc_final(ho3, c1w, c1b[None, :], c2w, c2b[None, :])


# speedup vs baseline: 61.7674x; 1.1350x over previous
import functools

import jax
import jax.numpy as jnp
from jax import lax
from jax.experimental import pallas as pl
from jax.experimental.pallas import tpu as pltpu
from jax.experimental.pallas import tpu_sc as plsc

N = 10000
E = 320000
HID = 64
HEADS = 8
NH = HEADS * HID
NEG_SLOPE = 0.2

NC = 2
NS = 16
NW = NC * NS
EW = E // NW
C1 = 80
C2 = 80
NCH1 = EW // C1
NCH2 = EW // C2
BN = 1000

_MESH = plsc.VectorSubcoreMesh(
    core_axis_name="c", subcore_axis_name="s", num_cores=NC, num_subcores=NS)


def _sc_pass1(ab_hbm, ba_hbm, gs_hbm, src_hbm, dst_hbm,
              denom_out, v_out,
              gs_v, denom_t,
              srcA, dstA, r1A, r2A, vbA, semIA, semGA,
              srcB, dstB, r1B, r2B, vbB, semIB, semGB):
    c_idx = lax.axis_index("c")
    s_idx = lax.axis_index("s")
    wid = s_idx * NC + c_idx
    iota = lax.iota(jnp.int32, 16)
    headmask = iota < 8

    pltpu.sync_copy(gs_hbm, gs_v)

    @plsc.parallel_loop(0, N * HEADS // 16, unroll=8)
    def zero(i):
        denom_t[pl.ds(i * 16, 16)] = jnp.zeros((16,), jnp.float32)

    def issue_idx(g, s_v, d_v, semI):
        e0 = wid * EW + g * C1
        pltpu.async_copy(src_hbm.at[pl.ds(e0, C1)], s_v, semI)
        pltpu.async_copy(dst_hbm.at[pl.ds(e0, C1)], d_v, semI)

    def wait_idx(s_v, d_v, semI):
        pltpu.make_async_copy(src_hbm.at[pl.ds(0, C1)], s_v, semI).wait()
        pltpu.make_async_copy(dst_hbm.at[pl.ds(0, C1)], d_v, semI).wait()

    def issue_g(s_v, d_v, r1, r2, semG):
        pltpu.async_copy(ab_hbm.at[d_v], r1, semG)
        pltpu.async_copy(ba_hbm.at[s_v], r2, semG)

    def wait_g(s_v, d_v, r1, r2, semG):
        pltpu.make_async_copy(ab_hbm.at[d_v], r1, semG).wait()
        pltpu.make_async_copy(ba_hbm.at[s_v], r2, semG).wait()

    def compute(g, s_v, d_v, r1, r2, vbuf):
        @plsc.parallel_loop(0, C1, unroll=4)
        def edge(e):
            s = r1[e, :] + r2[e, :]
            u = jnp.exp(jnp.maximum(s, NEG_SLOPE * s))
            esplat = jnp.full((16,), e, jnp.int32)
            srcsplat = plsc.load_gather(s_v, [esplat])
            gsv = plsc.load_gather(gs_v, [srcsplat])
            vbuf[e, :] = u * (1.0 + gsv)
            dstsplat = plsc.load_gather(d_v, [esplat])
            plsc.addupdate_scatter(denom_t, [dstsplat * 8 + iota], u,
                                   mask=headmask)

        e0 = wid * EW + g * C1
        pltpu.sync_copy(vbuf, v_out.at[pl.ds(e0, C1)])

    A = (srcA, dstA, r1A, r2A, vbA, semIA, semGA)
    B = (srcB, dstB, r1B, r2B, vbB, semIB, semGB)

    def _issue_idx(S, g):
        issue_idx(g, S[0], S[1], S[5])

    def _wait_idx(S):
        wait_idx(S[0], S[1], S[5])

    def _issue_g(S):
        issue_g(S[0], S[1], S[2], S[3], S[6])

    def _wait_g(S):
        wait_g(S[0], S[1], S[2], S[3], S[6])

    def _compute(S, g):
        compute(g, S[0], S[1], S[2], S[3], S[4])

    _issue_idx(A, 0)
    _wait_idx(A)
    _issue_g(A)

    def pair(p, _):
        g0 = 2 * p
        _issue_idx(B, g0 + 1)
        _wait_g(A)
        _wait_idx(B)
        _issue_g(B)
        _compute(A, g0)
        _issue_idx(A, g0 + 2)
        _wait_g(B)
        _wait_idx(A)
        _issue_g(A)
        _compute(B, g0 + 1)
        return _

    lax.fori_loop(0, (NCH1 - 1) // 2, pair, None)
    _wait_g(A)
    _compute(A, NCH1 - 1)
    pltpu.sync_copy(denom_t, denom_out.at[wid])


def _sc_pass2(hf_hbm, v_hbm, d16_hbm, src_hbm, dst_hbm, z64_hbm,
              acc_out,
              acc_sh,
              srcA, dstA, vbA, drA, hfA, cbA, msgA, semIA, semGA,
              srcB, dstB, vbB, drB, hfB, cbB, msgB, semIB, semGB):
    c_idx = lax.axis_index("c")
    s_idx = lax.axis_index("s")
    wid = s_idx * NC + c_idx
    iota = lax.iota(jnp.int32, 16)

    pltpu.sync_copy(z64_hbm.at[pl.ds(s_idx * 625, 625)],
                    acc_sh.at[pl.ds(s_idx * 625, 625)])
    plsc.subcore_barrier()

    def issue_idx(g, s_v, d_v, semI):
        e0 = wid * EW + g * C2
        pltpu.async_copy(src_hbm.at[pl.ds(e0, C2)], s_v, semI)
        pltpu.async_copy(dst_hbm.at[pl.ds(e0, C2)], d_v, semI)

    def wait_idx(s_v, d_v, semI):
        pltpu.make_async_copy(src_hbm.at[pl.ds(0, C2)], s_v, semI).wait()
        pltpu.make_async_copy(dst_hbm.at[pl.ds(0, C2)], d_v, semI).wait()

    def issue_g(g, s_v, d_v, vbuf, dr, hfb, semG):
        e0 = wid * EW + g * C2
        pltpu.async_copy(hf_hbm.at[s_v], hfb, semG)
        pltpu.async_copy(v_hbm.at[pl.ds(e0, C2)], vbuf, semG)
        pltpu.async_copy(d16_hbm.at[d_v], dr, semG)

    def wait_g(s_v, d_v, vbuf, dr, hfb, semG):
        pltpu.make_async_copy(hf_hbm.at[s_v], hfb, semG).wait()
        pltpu.make_async_copy(v_hbm.at[pl.ds(0, C2)], vbuf, semG).wait()
        pltpu.make_async_copy(d16_hbm.at[d_v], dr, semG).wait()

    def compute(s_v, d_v, vbuf, dr, hfb, cbuf, msg):
        @plsc.parallel_loop(0, C2, unroll=4)
        def coeff(e):
            vv = vbuf[e, :]
            dv = dr[e, :]
            c = vv / ((dv + 1e-16) * float(HEADS))
            cbuf[pl.ds(e * 16, 16)] = c

        @plsc.parallel_loop(0, C2, unroll=2)
        def edge(e):
            m = [jnp.zeros((16,), jnp.float32) for _ in range(4)]
            for h in range(HEADS):
                cb = plsc.load_gather(
                    cbuf, [jnp.full((16,), e * 16 + h, jnp.int32)])
                for jj in range(2):
                    seg = hfb[e, pl.ds(h * HID + jj * 32, 32)]
                    ev, od = plsc.unpack(
                        seg, format=plsc.PackFormat.INTERLEAVED,
                        preferred_element_type=jnp.float32)
                    m[2 * jj] = m[2 * jj] + cb * ev
                    m[2 * jj + 1] = m[2 * jj + 1] + cb * od
            for j in range(4):
                msg[e, pl.ds(j * 16, 16)] = m[j]

        pltpu.sync_copy(msg, acc_sh.at[d_v], add=True)

    A = (srcA, dstA, vbA, drA, hfA, cbA, msgA, semIA, semGA)
    B = (srcB, dstB, vbB, drB, hfB, cbB, msgB, semIB, semGB)

    def _issue_idx(S, g):
        issue_idx(g, S[0], S[1], S[7])

    def _wait_idx(S):
        wait_idx(S[0], S[1], S[7])

    def _issue_g(S, g):
        issue_g(g, S[0], S[1], S[2], S[3], S[4], S[8])

    def _wait_g(S):
        wait_g(S[0], S[1], S[2], S[3], S[4], S[8])

    def _compute(S):
        compute(S[0], S[1], S[2], S[3], S[4], S[5], S[6])

    _issue_idx(A, 0)
    _wait_idx(A)
    _issue_g(A, 0)

    def pair(p, _):
        g0 = 2 * p
        _issue_idx(B, g0 + 1)
        _wait_g(A)
        _wait_idx(B)
        _issue_g(B, g0 + 1)
        _compute(A)
        _issue_idx(A, g0 + 2)
        _wait_g(B)
        _wait_idx(A)
        _issue_g(A, g0 + 2)
        _compute(B)
        return _

    lax.fori_loop(0, (NCH2 - 1) // 2, pair, None)
    if NCH2 % 2 == 0:
        _issue_idx(B, NCH2 - 1)
        _wait_g(A)
        _wait_idx(B)
        _issue_g(B, NCH2 - 1)
        _compute(A)
        _wait_g(B)
        _compute(B)
    else:
        _wait_g(A)
        _compute(A)
    plsc.subcore_barrier()
    pltpu.sync_copy(acc_sh.at[pl.ds(s_idx * 625, 625)],
                    acc_out.at[c_idx, pl.ds(s_idx * 625, 625)])


def _make_pass1():
    return functools.partial(
        pl.kernel,
        out_type=[jax.ShapeDtypeStruct((NW, N * HEADS), jnp.float32),
                  jax.ShapeDtypeStruct((E, 16), jnp.float32)],
        mesh=_MESH,
        scratch_types=[
            pltpu.VMEM((N,), jnp.float32),
            pltpu.VMEM((N * HEADS,), jnp.float32),
        ] + 2 * [
            pltpu.VMEM((C1,), jnp.int32),
            pltpu.VMEM((C1,), jnp.int32),
            pltpu.VMEM((C1, 16), jnp.float32),
            pltpu.VMEM((C1, 16), jnp.float32),
            pltpu.VMEM((C1, 16), jnp.float32),
            pltpu.SemaphoreType.DMA,
            pltpu.SemaphoreType.DMA,
        ],
        compiler_params=pltpu.CompilerParams(needs_layout_passes=False, use_tc_tiling_on_sc=False),
    )(_sc_pass1)


def _make_pass2():
    return functools.partial(
        pl.kernel,
        out_type=[jax.ShapeDtypeStruct((NC, N, HID), jnp.float32)],
        mesh=_MESH,
        scratch_types=[
            pltpu.VMEM_SHARED((N, HID), jnp.float32),
        ] + 2 * [
            pltpu.VMEM((C2,), jnp.int32),
            pltpu.VMEM((C2,), jnp.int32),
            pltpu.VMEM((C2, 16), jnp.float32),
            pltpu.VMEM((C2, 16), jnp.float32),
            pltpu.VMEM((C2, NH), jnp.bfloat16),
            pltpu.VMEM((C2 * 16,), jnp.float32),
            pltpu.VMEM((C2, HID), jnp.float32),
            pltpu.SemaphoreType.DMA,
            pltpu.SemaphoreType.DMA,
        ],
        compiler_params=pltpu.CompilerParams(needs_layout_passes=False, use_tc_tiling_on_sc=False),
    )(_sc_pass2)


def _tc_pre_kernel(x_ref, w1_ref, bs_ref, bd_ref, gs_ref, gewt_ref, geb_ref,
                   h_ref, ab_ref, ba_ref, gemb_ref):
    h = lax.dot_general(x_ref[...], w1_ref[...], (((1,), (1,)), ((), ())),
                        preferred_element_type=jnp.float32)
    h_ref[...] = h.astype(jnp.bfloat16)
    a1 = jnp.dot(h, bs_ref[...], preferred_element_type=jnp.float32)
    a2 = jnp.dot(h, bd_ref[...], preferred_element_type=jnp.float32)
    ab_ref[...] = jnp.concatenate([a1, a2], axis=-1)
    ba_ref[...] = jnp.concatenate([a2, a1], axis=-1)
    gemb_ref[...] = jnp.dot(gs_ref[...], gewt_ref[...],
                            preferred_element_type=jnp.float32) + geb_ref[...]


def _tc_dcomb_kernel(dp_ref, d16_ref):
    d = jnp.sum(dp_ref[...], axis=0)
    d16_ref[...] = jnp.concatenate([d, jnp.ones_like(d)], axis=-1)


def _tc_dcomb(denom_p):
    return pl.pallas_call(
        _tc_dcomb_kernel,
        grid=(N // BN,),
        in_specs=[pl.BlockSpec((NW, BN, HEADS), lambda i: (0, i, 0))],
        out_specs=pl.BlockSpec((BN, 16), lambda i: (i, 0)),
        out_shape=jax.ShapeDtypeStruct((N, 16), jnp.float32),
    )(denom_p)


def _tc_mid_kernel(acc_ref, gemb_ref, xp_ref, pw_ref, pb_ref, ng_ref, nb_ref,
                   wn_ref, bs_ref, bd_ref, pm_ref,
                   ho_ref, hn_ref, ab_ref, ba_ref):
    s = lax.dot_general(acc_ref[0] + acc_ref[1], pm_ref[...],
                        (((1,), (0,)), ((), ())),
                        preferred_element_type=jnp.float32) + gemb_ref[...]
    mu = jnp.mean(s, axis=-1, keepdims=True)
    var = jnp.mean((s - mu) ** 2, axis=-1, keepdims=True)
    ln = (s - mu) * lax.rsqrt(var + 1e-5) * ng_ref[...] + nb_ref[...]
    ident = lax.dot_general(xp_ref[...], pw_ref[...], (((1,), (1,)), ((), ())),
                            preferred_element_type=jnp.float32) + pb_ref[...]
    ho = jnp.maximum(ln + ident, 0.0)
    ho_ref[...] = ho
    hn = lax.dot_general(ho, wn_ref[...], (((1,), (1,)), ((), ())),
                         preferred_element_type=jnp.float32)
    hn_ref[...] = hn.astype(jnp.bfloat16)
    a1 = jnp.dot(hn, bs_ref[...], preferred_element_type=jnp.float32)
    a2 = jnp.dot(hn, bd_ref[...], preferred_element_type=jnp.float32)
    ab_ref[...] = jnp.concatenate([a1, a2], axis=-1)
    ba_ref[...] = jnp.concatenate([a2, a1], axis=-1)


def _tc_last_kernel(acc_ref, gemb_ref, xp_ref, pw_ref, pb_ref, ng_ref, nb_ref,
                    pm_ref, ho_ref):
    s = lax.dot_general(acc_ref[0] + acc_ref[1], pm_ref[...],
                        (((1,), (0,)), ((), ())),
                        preferred_element_type=jnp.float32) + gemb_ref[...]
    mu = jnp.mean(s, axis=-1, keepdims=True)
    var = jnp.mean((s - mu) ** 2, axis=-1, keepdims=True)
    ln = (s - mu) * lax.rsqrt(var + 1e-5) * ng_ref[...] + nb_ref[...]
    ident = lax.dot_general(xp_ref[...], pw_ref[...], (((1,), (1,)), ((), ())),
                            preferred_element_type=jnp.float32) + pb_ref[...]
    ho_ref[...] = jnp.maximum(ln + ident, 0.0)


def _tc_final_kernel(h_ref, c1w_ref, c1b_ref, c2w_ref, c2b_ref, o_ref):
    pooled = jnp.mean(h_ref[...], axis=0, keepdims=True)
    z = lax.dot_general(pooled, c1w_ref[...], (((1,), (1,)), ((), ())),
                        preferred_element_type=jnp.float32) + c1b_ref[...]
    z = jnp.maximum(z, 0.0)
    z = jnp.sum(z * c2w_ref[...], axis=-1, keepdims=True) + c2b_ref[...]
    o_ref[...] = 1.0 / (1.0 + jnp.exp(-z))


def _full(shape):
    return pl.BlockSpec(shape, lambda *_: tuple(0 for _ in shape))


def _tc_pre(x, w1, bs, bd, gs2d, gewt, geb2d):
    din = x.shape[1]
    return pl.pallas_call(
        _tc_pre_kernel,
        grid=(N // BN,),
        in_specs=[
            pl.BlockSpec((BN, din), lambda i: (i, 0)),
            _full((NH, din)), _full((NH, HEADS)), _full((NH, HEADS)),
            pl.BlockSpec((BN, 1), lambda i: (i, 0)),
            _full((1, HID)), _full((1, HID)),
        ],
        out_specs=[
            pl.BlockSpec((BN, NH), lambda i: (i, 0)),
            pl.BlockSpec((BN, 16), lambda i: (i, 0)),
            pl.BlockSpec((BN, 16), lambda i: (i, 0)),
            pl.BlockSpec((BN, HID), lambda i: (i, 0)),
        ],
        out_shape=[
            jax.ShapeDtypeStruct((N, NH), jnp.bfloat16),
            jax.ShapeDtypeStruct((N, 16), jnp.float32),
            jax.ShapeDtypeStruct((N, 16), jnp.float32),
            jax.ShapeDtypeStruct((N, HID), jnp.float32),
        ],
    )(x, w1, bs, bd, gs2d, gewt, geb2d)


def _tc_mid(accp, gemb, xp, pw, pb2d, ng2d, nb2d, wn, bs, bd, pmat):
    dp = xp.shape[1]
    return pl.pallas_call(
        _tc_mid_kernel,
        grid=(N // BN,),
        in_specs=[
            pl.BlockSpec((NC, BN, HID), lambda i: (0, i, 0)),
            pl.BlockSpec((BN, HID), lambda i: (i, 0)),
            pl.BlockSpec((BN, dp), lambda i: (i, 0)),
            _full((HID, dp)), _full((1, HID)), _full((1, HID)), _full((1, HID)),
            _full((NH, HID)), _full((NH, HEADS)), _full((NH, HEADS)),
            _full((HID, HID)),
        ],
        out_specs=[
            pl.BlockSpec((BN, HID), lambda i: (i, 0)),
            pl.BlockSpec((BN, NH), lambda i: (i, 0)),
            pl.BlockSpec((BN, 16), lambda i: (i, 0)),
            pl.BlockSpec((BN, 16), lambda i: (i, 0)),
        ],
        out_shape=[
            jax.ShapeDtypeStruct((N, HID), jnp.float32),
            jax.ShapeDtypeStruct((N, NH), jnp.bfloat16),
            jax.ShapeDtypeStruct((N, 16), jnp.float32),
            jax.ShapeDtypeStruct((N, 16), jnp.float32),
        ],
    )(accp, gemb, xp, pw, pb2d, ng2d, nb2d, wn, bs, bd, pmat)


def _tc_last(accp, gemb, xp, pw, pb2d, ng2d, nb2d, pmat):
    dp = xp.shape[1]
    return pl.pallas_call(
        _tc_last_kernel,
        grid=(N // BN,),
        in_specs=[
            pl.BlockSpec((NC, BN, HID), lambda i: (0, i, 0)),
            pl.BlockSpec((BN, HID), lambda i: (i, 0)),
            pl.BlockSpec((BN, dp), lambda i: (i, 0)),
            _full((HID, dp)), _full((1, HID)), _full((1, HID)), _full((1, HID)),
            _full((HID, HID)),
        ],
        out_specs=pl.BlockSpec((BN, HID), lambda i: (i, 0)),
        out_shape=jax.ShapeDtypeStruct((N, HID), jnp.float32),
    )(accp, gemb, xp, pw, pb2d, ng2d, nb2d, pmat)


def _tc_final(h3, c1w, c1b2d, c2w, c2b2d):
    return pl.pallas_call(
        _tc_final_kernel,
        in_specs=[_full((N, HID)), _full((HID // 2, HID)),
                  _full((1, HID // 2)), _full((1, HID // 2)), _full((1, 1))],
        out_specs=_full((1, 1)),
        out_shape=jax.ShapeDtypeStruct((1, 1), jnp.float32),
    )(h3, c1w, c1b2d, c2w, c2b2d)


def _block_diag(att):
    eye = jnp.eye(HEADS, dtype=jnp.float32)
    return jnp.einsum('hd,hg->hdg', att[0], eye).reshape(NH, HEADS)


def kernel(x, edge_index, gambling_scores, W1, as1, ad1, W2, as2, ad2, W3, as3, ad3, p1w, p1b, p2w, p2b, p3w, p3b, n1g, n1b, n2g, n2b, n3g, n3b, gew, geb, c1w, c1b, c2w, c2b):
    src = edge_index[0]
    dst = edge_index[1]
    gs = gambling_scores
    z64 = jnp.zeros((N, HID), jnp.float32)
    gs2d = gs[:, None]
    gewt = gew.T
    geb2d = geb[None, :]
    perm = jnp.array([g * 32 + p * 2 + o
                      for g in range(2) for o in range(2) for p in range(16)],
                     jnp.int32)
    pmat = jnp.eye(HID, dtype=jnp.float32)[perm]

    pass1 = _make_pass1()
    pass2 = _make_pass2()

    def layer_sparse(hf, ab, ba):
        denom_p, v = pass1(ab, ba, gs, src, dst)
        d16 = _tc_dcomb(denom_p.reshape(NW, N, HEADS))
        acc_p, = pass2(hf, v, d16, src, dst, z64)
        return acc_p

    h1, ab1, ba1, gemb = _tc_pre(x, W1, _block_diag(as1), _block_diag(ad1),
                                 gs2d, gewt, geb2d)
    acc1 = layer_sparse(h1, ab1, ba1)
    ho1, h2, ab2, ba2 = _tc_mid(acc1, gemb, x, p1w, p1b[None, :],
                                n1g[None, :], n1b[None, :], W2,
                                _block_diag(as2), _block_diag(ad2), pmat)
    acc2 = layer_sparse(h2, ab2, ba2)
    ho2, h3, ab3, ba3 = _tc_mid(acc2, gemb, ho1, p2w, p2b[None, :],
                                n2g[None, :], n2b[None, :], W3,
                                _block_diag(as3), _block_diag(ad3), pmat)
    acc3 = layer_sparse(h3, ab3, ba3)
    ho3 = _tc_last(acc3, gemb, ho2, p3w, p3b[None, :],
                   n3g[None, :], n3b[None, :], pmat)
    return _tc_final(ho3, c1w, c1b[None, :], c2w, c2b[None, :])
